# baseline (speedup 1.0000x reference)
import functools

import jax
import jax.numpy as jnp
from jax.experimental import pallas as pl
from jax.experimental.pallas import tpu as pltpu

_T = 5
_G = 4
_EPS = 1e-5


def _oni_kernel(x_ref, o_ref, cache_ref, acc_ref, cs_ref, m_ref, off_ref,
                *, g, nc, d, n_samples):
    i = pl.program_id(0)
    j = pl.program_id(1)
    p = jax.lax.rem(i, 2)

    @pl.when(jnp.logical_and(i < g, j == 0))
    def _():
        acc_ref[...] = jnp.zeros_like(acc_ref)
        cs_ref[...] = jnp.zeros_like(cs_ref)

    @pl.when(jnp.logical_and(i < g, j < nc))
    def _cov_phase():
        xb = x_ref[...].astype(jnp.bfloat16)
        cache_ref[p, j] = xb
        n_panels = xb.shape[1] // d
        acc = acc_ref[...]
        cs = cs_ref[...]
        for k in range(n_panels):
            panel = xb[:, k * d:(k + 1) * d]
            acc += jax.lax.dot_general(
                panel, panel, (((0,), (0,)), ((), ())),
                preferred_element_type=jnp.float32)
            cs += jnp.sum(panel.astype(jnp.float32), axis=0, keepdims=True)
        acc_ref[...] = acc
        cs_ref[...] = cs

    @pl.when(jnp.logical_and(i < g, j == nc))
    def _ns_epilogue():
        s_raw = acc_ref[...]
        mu = cs_ref[...] * (1.0 / n_samples)
        outer = jax.lax.dot_general(
            mu, mu, (((0,), (0,)), ((), ())),
            preferred_element_type=jnp.float32)
        rows = jax.lax.broadcasted_iota(jnp.int32, (d, d), 0)
        cols = jax.lax.broadcasted_iota(jnp.int32, (d, d), 1)
        eye = jnp.where(rows == cols, jnp.float32(1.0), jnp.float32(0.0))
        s = s_raw - n_samples * outer + _EPS * eye
        norm = jnp.sqrt(jnp.sum(s * s))
        s = s * (1.0 / norm)
        b = 1.5 * eye - 0.5 * s
        s_b = s.astype(jnp.bfloat16)
        for _ in range(_T - 1):
            b_b = b.astype(jnp.bfloat16)
            b2 = jnp.dot(b_b, b_b, preferred_element_type=jnp.float32)
            bs = jnp.dot(b_b, s_b, preferred_element_type=jnp.float32)
            b = 1.5 * b - 0.5 * jnp.dot(
                b2.astype(jnp.bfloat16), bs.astype(jnp.bfloat16),
                preferred_element_type=jnp.float32)
        m = b.T * jax.lax.rsqrt(norm)
        m_ref[p] = m.astype(jnp.bfloat16)
        off_ref[p] = jnp.dot(mu, m, preferred_element_type=jnp.float32)

    @pl.when(jnp.logical_and(i > 0, j < nc))
    def _apply_phase():
        q = 1 - p
        xb = cache_ref[q, j]
        n_panels = xb.shape[1] // d
        m = m_ref[q]
        off = off_ref[q]
        for k in range(n_panels):
            panel = xb[:, k * d:(k + 1) * d]
            o_ref[:, k * d:(k + 1) * d] = jnp.dot(
                panel, m, preferred_element_type=jnp.float32) - off


def kernel(inputs):
    rows, c = inputs.shape
    g = _G
    d = c // g
    band = rows // g
    n_samples = band * (c // d)

    r = 512
    nc = band // r

    out = pl.pallas_call(
        functools.partial(_oni_kernel, g=g, nc=nc, d=d, n_samples=n_samples),
        grid=(g + 1, nc + 1),
        in_specs=[
            pl.BlockSpec(
                (r, c),
                lambda i, j: (jnp.where(
                    jnp.logical_and(i < g, j < nc),
                    i * nc + j,
                    jnp.minimum(i, g - 1) * nc + nc - 1), 0)),
        ],
        out_specs=pl.BlockSpec(
            (r, c),
            lambda i, j: (jnp.where(
                i > 0, (i - 1) * nc + jnp.minimum(j, nc - 1), 0), 0)),
        out_shape=jax.ShapeDtypeStruct(inputs.shape, jnp.float32),
        scratch_shapes=[
            pltpu.VMEM((2, nc, r, c), jnp.bfloat16),
            pltpu.VMEM((d, d), jnp.float32),
            pltpu.VMEM((1, d), jnp.float32),
            pltpu.VMEM((2, d, d), jnp.bfloat16),
            pltpu.VMEM((2, 1, d), jnp.float32),
        ],
        compiler_params=pltpu.CompilerParams(
            dimension_semantics=("arbitrary", "arbitrary"),
            vmem_limit_bytes=60 * 1024 * 1024),
        name="oni_fused",
    )(inputs)

    return out


# speedup vs baseline: 1.0167x; 1.0167x over previous
import functools

import jax
import jax.numpy as jnp
from jax.experimental import pallas as pl
from jax.experimental.pallas import tpu as pltpu

_T = 5
_G = 4
_EPS = 1e-5


def _oni_kernel(x_ref, o_ref, cache_ref, acc_ref, cs_ref, m_ref, off_ref,
                *, g, nc, d, n_samples):
    i = pl.program_id(0)
    j = pl.program_id(1)
    p = jax.lax.rem(i, 2)

    @pl.when(jnp.logical_and(i < g, j == 0))
    def _():
        acc_ref[...] = jnp.zeros_like(acc_ref)
        cs_ref[...] = jnp.zeros_like(cs_ref)

    @pl.when(i < g)
    def _cov_phase():
        xb = x_ref[...].astype(jnp.bfloat16)
        cache_ref[p, j] = xb
        n_panels = xb.shape[1] // d
        acc = acc_ref[...]
        cs = cs_ref[...]
        for k in range(n_panels):
            panel = xb[:, k * d:(k + 1) * d]
            acc += jax.lax.dot_general(
                panel, panel, (((0,), (0,)), ((), ())),
                preferred_element_type=jnp.float32)
            cs += jnp.sum(panel.astype(jnp.float32), axis=0, keepdims=True)
        acc_ref[...] = acc
        cs_ref[...] = cs

        @pl.when(j == nc - 1)
        def _ns_epilogue():
            s_raw = acc_ref[...]
            mu = cs_ref[...] * (1.0 / n_samples)
            outer = jax.lax.dot_general(
                mu, mu, (((0,), (0,)), ((), ())),
                preferred_element_type=jnp.float32)
            rows = jax.lax.broadcasted_iota(jnp.int32, (d, d), 0)
            cols = jax.lax.broadcasted_iota(jnp.int32, (d, d), 1)
            eye = jnp.where(rows == cols, jnp.float32(1.0), jnp.float32(0.0))
            s = s_raw - n_samples * outer + _EPS * eye
            norm = jnp.sqrt(jnp.sum(s * s))
            s = s * (1.0 / norm)
            b = 1.5 * eye - 0.5 * s
            s_b = s.astype(jnp.bfloat16)
            for _ in range(_T - 1):
                b_b = b.astype(jnp.bfloat16)
                b2 = jnp.dot(b_b, b_b, preferred_element_type=jnp.float32)
                b3 = jnp.dot(b2.astype(jnp.bfloat16), b_b,
                             preferred_element_type=jnp.float32)
                b = 1.5 * b - 0.5 * jnp.dot(
                    b3.astype(jnp.bfloat16), s_b,
                    preferred_element_type=jnp.float32)
            m = b.T * jax.lax.rsqrt(norm)
            m_ref[p] = m.astype(jnp.bfloat16)
            off_ref[p] = jnp.dot(mu, m, preferred_element_type=jnp.float32)

    @pl.when(i > 0)
    def _apply_phase():
        q = 1 - p
        xb = cache_ref[q, j]
        n_panels = xb.shape[1] // d
        m = m_ref[q]
        off = off_ref[q]
        for k in range(n_panels):
            panel = xb[:, k * d:(k + 1) * d]
            o_ref[:, k * d:(k + 1) * d] = jnp.dot(
                panel, m, preferred_element_type=jnp.float32) - off


def kernel(inputs):
    rows, c = inputs.shape
    g = _G
    d = c // g
    band = rows // g
    n_samples = band * (c // d)

    r = 512
    nc = band // r

    out = pl.pallas_call(
        functools.partial(_oni_kernel, g=g, nc=nc, d=d, n_samples=n_samples),
        grid=(g + 1, nc),
        in_specs=[
            pl.BlockSpec(
                (r, c),
                lambda i, j: (jnp.where(i < g, i * nc + j, g * nc - 1), 0)),
        ],
        out_specs=pl.BlockSpec(
            (r, c),
            lambda i, j: (jnp.where(i > 0, (i - 1) * nc + j, 0), 0)),
        out_shape=jax.ShapeDtypeStruct(inputs.shape, jnp.float32),
        scratch_shapes=[
            pltpu.VMEM((2, nc, r, c), jnp.bfloat16),
            pltpu.VMEM((d, d), jnp.float32),
            pltpu.VMEM((1, d), jnp.float32),
            pltpu.VMEM((2, d, d), jnp.bfloat16),
            pltpu.VMEM((2, 1, d), jnp.float32),
        ],
        compiler_params=pltpu.CompilerParams(
            dimension_semantics=("arbitrary", "arbitrary"),
            vmem_limit_bytes=60 * 1024 * 1024),
        name="oni_fused",
    )(inputs)

    return out
